# baseline (speedup 1.0000x reference)
import jax
import jax.numpy as jnp
from jax.experimental import pallas as pl
from jax.experimental.pallas import tpu as pltpu


def _rup8(n):
    return (n + 7) & ~7


def _slab_offsets(d_in, h):
    o_w1 = 0
    o_b1 = _rup8(o_w1 + d_in)
    o_wih = _rup8(o_b1 + 1)
    o_whh = o_wih + _rup8(h)
    o_bl = o_whh + _rup8(h)
    main_rows = _rup8(o_bl + 1)
    e_w2 = 0
    e_b2 = _rup8(e_w2 + h)
    e_w3 = _rup8(e_b2 + 1)
    e_b3 = _rup8(e_w3 + 1)
    epi_rows = _rup8(e_b3 + 1)
    return dict(o_w1=o_w1, o_b1=o_b1, o_wih=o_wih, o_whh=o_whh, o_bl=o_bl,
                main_rows=main_rows, e_w2=e_w2, e_b2=e_b2, e_w3=e_w3,
                e_b3=e_b3, epi_rows=epi_rows)


def _fused_kernel(s_ref, a_ref, la_ref, h0_hbm, c0_hbm, epi_hbm, main_hbm,
                  q_ref, hn_ref, cn_ref,
                  xtm_buf, pre_buf, w1_buf, wih_buf, whh_buf,
                  h_buf, c_buf, epi_buf, bl_buf,
                  sem1, sem2, sem3, sem4, sem5, sem6, sem0):
    B, T, sd = s_ref.shape
    ad = a_ref.shape[1]
    lad = la_ref.shape[1]
    d_in = sd + ad + lad
    H = h_buf.shape[1]
    L = _slab_offsets(d_in, H)
    main_rows_pad = L["main_rows"] - L["o_bl"]

    cp1 = pltpu.make_async_copy(
        main_hbm.at[pl.ds(0, L["o_wih"]), pl.ds(0, H)], w1_buf, sem1)
    nih = L["o_whh"] - L["o_wih"]
    nhh = L["o_bl"] - L["o_whh"]
    ck = nih // 4
    cp2s = [pltpu.make_async_copy(
        main_hbm.at[pl.ds(L["o_wih"] + i * ck, ck)],
        wih_buf.at[pl.ds(i * ck, ck)], sem2) for i in range(4)]
    ck3 = nhh // 4
    cp3s = [pltpu.make_async_copy(
        main_hbm.at[pl.ds(L["o_whh"] + i * ck3, ck3)],
        whh_buf.at[pl.ds(i * ck3, ck3)], sem3) for i in range(4)]
    cp0 = pltpu.make_async_copy(
        main_hbm.at[pl.ds(L["o_bl"], main_rows_pad)], bl_buf, sem0)
    cp4 = pltpu.make_async_copy(h0_hbm.at[0], h_buf, sem4)
    cp5 = pltpu.make_async_copy(c0_hbm.at[0], c_buf, sem5)
    cp6 = pltpu.make_async_copy(
        epi_hbm.at[pl.ds(0, L["epi_rows"]), pl.ds(0, H)], epi_buf, sem6)
    cp0.start()
    cp1.start()
    for cp in cp2s:
        cp.start()
    for cp in cp3s:
        cp.start()
    cp4.start()
    cp5.start()
    cp6.start()

    for t in range(T):
        xtm_buf[t * B:(t + 1) * B, 0:sd] = s_ref[:, t, :]
        xtm_buf[t * B:(t + 1) * B, sd:sd + ad] = a_ref[t].T
        xtm_buf[t * B:(t + 1) * B, sd + ad:d_in] = la_ref[t].T

    cp1.wait()
    w1 = w1_buf[0:d_in, :]
    b1 = w1_buf[L["o_b1"]:L["o_b1"] + 1, :]
    a1 = jnp.maximum(
        jnp.dot(xtm_buf[...], w1, preferred_element_type=jnp.float32) + b1,
        0.0)

    cp0.wait()
    for cp in cp2s:
        cp.wait()
    pre_buf[...] = jnp.dot(a1, wih_buf[0:H, :],
                           preferred_element_type=jnp.float32) + bl_buf[0:1, :]

    for cp in cp3s:
        cp.wait()
    w_hh = whh_buf[0:H, :]

    cp4.wait()
    cp5.wait()
    h = h_buf[...]
    c = c_buf[...]
    hs_steps = []
    for t in range(T):
        gt = pre_buf[t * B:(t + 1) * B, :] + jnp.dot(
            h, w_hh, preferred_element_type=jnp.float32)
        s_if = jax.nn.sigmoid(gt[:, 0:2 * H])
        o_g = jax.nn.sigmoid(gt[:, 3 * H:4 * H])
        g_g = jnp.tanh(gt[:, 2 * H:3 * H])
        c = s_if[:, H:2 * H] * c + s_if[:, 0:H] * g_g
        h = o_g * jnp.tanh(c)
        hs_steps.append(h)

    hn_ref[0] = h
    cn_ref[0] = c

    cp6.wait()
    w2 = epi_buf[0:H, :]
    b2 = epi_buf[L["e_b2"]:L["e_b2"] + 1, :]
    w3r = epi_buf[L["e_w3"]:L["e_w3"] + 1, :]
    b3 = epi_buf[L["e_b3"]:L["e_b3"] + 1, 0:1]
    hs = jnp.concatenate(hs_steps, axis=0)
    a2 = jnp.maximum(
        jnp.dot(hs, w2, preferred_element_type=jnp.float32) + b2, 0.0)
    q_ref[...] = jnp.dot(w3r, a2.T, preferred_element_type=jnp.float32) + b3


def kernel(main, epi, state, action, last_action, h0, c0):
    B, T, _ = state.shape
    H = h0.shape[-1]
    d_in = state.shape[-1] + action.shape[-1] + last_action.shape[-1]
    L = _slab_offsets(d_in, H)
    G = 4 * H

    a_tm = jnp.transpose(action, (1, 2, 0))
    la_tm = jnp.transpose(last_action, (1, 2, 0))

    flops = 2 * T * B * (d_in * H + 2 * H * 4 * H + H * H + H)
    nbytes = 4 * (T * B * d_in + 2 * B * H + L["o_wih"] * H
                  + (L["main_rows"] - L["o_wih"]) * G
                  + L["epi_rows"] * H + T * B + 2 * B * H)
    cost = pl.CostEstimate(flops=flops,
                           transcendentals=3 * T * B * H,
                           bytes_accessed=nbytes)

    q_row, h_n, c_n = pl.pallas_call(
        _fused_kernel,
        out_shape=(
            jax.ShapeDtypeStruct((1, T * B), jnp.float32),
            jax.ShapeDtypeStruct((1, B, H), jnp.float32),
            jax.ShapeDtypeStruct((1, B, H), jnp.float32),
        ),
        grid_spec=pltpu.PrefetchScalarGridSpec(
            num_scalar_prefetch=0,
            grid=(1,),
            in_specs=[
                pl.BlockSpec(state.shape, lambda i: (0, 0, 0)),
                pl.BlockSpec(a_tm.shape, lambda i: (0, 0, 0)),
                pl.BlockSpec(la_tm.shape, lambda i: (0, 0, 0)),
                pl.BlockSpec(memory_space=pl.ANY),
                pl.BlockSpec(memory_space=pl.ANY),
                pl.BlockSpec(memory_space=pl.ANY),
                pl.BlockSpec(memory_space=pl.ANY),
            ],
            out_specs=[
                pl.BlockSpec((1, T * B), lambda i: (0, 0)),
                pl.BlockSpec((1, B, H), lambda i: (0, 0, 0)),
                pl.BlockSpec((1, B, H), lambda i: (0, 0, 0)),
            ],
            scratch_shapes=[
                pltpu.VMEM((T * B, d_in), jnp.float32),
                pltpu.VMEM((T * B, G), jnp.float32),
                pltpu.VMEM((L["o_wih"], H), jnp.float32),
                pltpu.VMEM((L["o_whh"] - L["o_wih"], G), jnp.float32),
                pltpu.VMEM((L["o_bl"] - L["o_whh"], G), jnp.float32),
                pltpu.VMEM((B, H), jnp.float32),
                pltpu.VMEM((B, H), jnp.float32),
                pltpu.VMEM((L["epi_rows"], H), jnp.float32),
                pltpu.VMEM((L["main_rows"] - L["o_bl"], G), jnp.float32),
                pltpu.SemaphoreType.DMA,
                pltpu.SemaphoreType.DMA,
                pltpu.SemaphoreType.DMA,
                pltpu.SemaphoreType.DMA,
                pltpu.SemaphoreType.DMA,
                pltpu.SemaphoreType.DMA,
                pltpu.SemaphoreType.DMA,
            ],
        ),
        compiler_params=pltpu.CompilerParams(
            dimension_semantics=("arbitrary",),
        ),
        cost_estimate=cost,
    )(state, a_tm, la_tm, h0, c0, epi, main)

    q = q_row.reshape(T, B).T[..., None]
    return q, (h_n, c_n)


# speedup vs baseline: 1.0246x; 1.0246x over previous
import jax
import jax.numpy as jnp
from jax.experimental import pallas as pl
from jax.experimental.pallas import tpu as pltpu


def _rup8(n):
    return (n + 7) & ~7


def _slab_offsets(d_in, h):
    o_w1 = 0
    o_b1 = _rup8(o_w1 + d_in)
    o_wih = _rup8(o_b1 + 1)
    o_whh = o_wih + _rup8(h)
    o_bl = o_whh + _rup8(h)
    main_rows = _rup8(o_bl + 1)
    e_w2 = 0
    e_b2 = _rup8(e_w2 + h)
    e_w3 = _rup8(e_b2 + 1)
    e_b3 = _rup8(e_w3 + 1)
    epi_rows = _rup8(e_b3 + 1)
    return dict(o_w1=o_w1, o_b1=o_b1, o_wih=o_wih, o_whh=o_whh, o_bl=o_bl,
                main_rows=main_rows, e_w2=e_w2, e_b2=e_b2, e_w3=e_w3,
                e_b3=e_b3, epi_rows=epi_rows)


def _fused_kernel(s_ref, a_ref, la_ref, h0_hbm, c0_hbm, epi_hbm, main_hbm,
                  q_ref, hn_ref, cn_ref,
                  xtm_buf, pre_buf, w1_buf, wih_buf, whh_buf,
                  h_buf, c_buf, epi_buf, bl_buf,
                  sem1, sem2, sem2b, sem3, sem3b, sem4, sem5, sem6, sem0):
    B, T, sd = s_ref.shape
    ad = a_ref.shape[1]
    lad = la_ref.shape[1]
    d_in = sd + ad + lad
    H = h_buf.shape[1]
    L = _slab_offsets(d_in, H)
    main_rows_pad = L["main_rows"] - L["o_bl"]

    cp1 = pltpu.make_async_copy(
        main_hbm.at[pl.ds(0, L["o_wih"]), pl.ds(0, H)], w1_buf, sem1)
    nih = L["o_whh"] - L["o_wih"]
    nhh = L["o_bl"] - L["o_whh"]
    cp2 = pltpu.make_async_copy(
        main_hbm.at[pl.ds(L["o_wih"], nih // 2)],
        wih_buf.at[pl.ds(0, nih // 2)], sem2)
    cp2b = pltpu.make_async_copy(
        main_hbm.at[pl.ds(L["o_wih"] + nih // 2, nih - nih // 2)],
        wih_buf.at[pl.ds(nih // 2, nih - nih // 2)], sem2b)
    cp3 = pltpu.make_async_copy(
        main_hbm.at[pl.ds(L["o_whh"], nhh // 2)],
        whh_buf.at[pl.ds(0, nhh // 2)], sem3)
    cp3b = pltpu.make_async_copy(
        main_hbm.at[pl.ds(L["o_whh"] + nhh // 2, nhh - nhh // 2)],
        whh_buf.at[pl.ds(nhh // 2, nhh - nhh // 2)], sem3b)
    cp0 = pltpu.make_async_copy(
        main_hbm.at[pl.ds(L["o_bl"], main_rows_pad)], bl_buf, sem0)
    cp4 = pltpu.make_async_copy(h0_hbm.at[0], h_buf, sem4)
    cp5 = pltpu.make_async_copy(c0_hbm.at[0], c_buf, sem5)
    cp6 = pltpu.make_async_copy(
        epi_hbm.at[pl.ds(0, L["epi_rows"]), pl.ds(0, H)], epi_buf, sem6)
    cp0.start()
    cp1.start()
    cp2.start()
    cp2b.start()
    cp3.start()
    cp3b.start()
    cp4.start()
    cp5.start()
    cp6.start()

    for t in range(T):
        xtm_buf[t * B:(t + 1) * B, 0:sd] = s_ref[:, t, :]
        xtm_buf[t * B:(t + 1) * B, sd:sd + ad] = a_ref[t].T
        xtm_buf[t * B:(t + 1) * B, sd + ad:d_in] = la_ref[t].T

    cp1.wait()
    w1 = w1_buf[0:d_in, :]
    b1 = w1_buf[L["o_b1"]:L["o_b1"] + 1, :]
    a1 = jnp.maximum(
        jnp.dot(xtm_buf[...], w1, preferred_element_type=jnp.float32) + b1,
        0.0)

    cp0.wait()
    cp2.wait()
    cp2b.wait()
    pre_buf[...] = jnp.dot(a1, wih_buf[0:H, :],
                           preferred_element_type=jnp.float32) + bl_buf[0:1, :]

    cp3.wait()
    cp3b.wait()
    w_hh = whh_buf[0:H, :]

    cp4.wait()
    cp5.wait()
    h = h_buf[...]
    c = c_buf[...]
    hs_steps = []
    for t in range(T):
        gt = pre_buf[t * B:(t + 1) * B, :] + jnp.dot(
            h, w_hh, preferred_element_type=jnp.float32)
        s_if = jax.nn.sigmoid(gt[:, 0:2 * H])
        o_g = jax.nn.sigmoid(gt[:, 3 * H:4 * H])
        g_g = jnp.tanh(gt[:, 2 * H:3 * H])
        c = s_if[:, H:2 * H] * c + s_if[:, 0:H] * g_g
        h = o_g * jnp.tanh(c)
        hs_steps.append(h)

    hn_ref[0] = h
    cn_ref[0] = c

    cp6.wait()
    w2 = epi_buf[0:H, :]
    b2 = epi_buf[L["e_b2"]:L["e_b2"] + 1, :]
    w3r = epi_buf[L["e_w3"]:L["e_w3"] + 1, :]
    b3 = epi_buf[L["e_b3"]:L["e_b3"] + 1, 0:1]
    hs = jnp.concatenate(hs_steps, axis=0)
    a2 = jnp.maximum(
        jnp.dot(hs, w2, preferred_element_type=jnp.float32) + b2, 0.0)
    q_ref[...] = jnp.dot(w3r, a2.T, preferred_element_type=jnp.float32) + b3


def kernel(main, epi, state, action, last_action, h0, c0):
    B, T, _ = state.shape
    H = h0.shape[-1]
    d_in = state.shape[-1] + action.shape[-1] + last_action.shape[-1]
    L = _slab_offsets(d_in, H)
    G = 4 * H

    a_tm = jnp.transpose(action, (1, 2, 0))
    la_tm = jnp.transpose(last_action, (1, 2, 0))

    flops = 2 * T * B * (d_in * H + 2 * H * 4 * H + H * H + H)
    nbytes = 4 * (T * B * d_in + 2 * B * H + L["o_wih"] * H
                  + (L["main_rows"] - L["o_wih"]) * G
                  + L["epi_rows"] * H + T * B + 2 * B * H)
    cost = pl.CostEstimate(flops=flops,
                           transcendentals=3 * T * B * H,
                           bytes_accessed=nbytes)

    q_row, h_n, c_n = pl.pallas_call(
        _fused_kernel,
        out_shape=(
            jax.ShapeDtypeStruct((1, T * B), jnp.float32),
            jax.ShapeDtypeStruct((1, B, H), jnp.float32),
            jax.ShapeDtypeStruct((1, B, H), jnp.float32),
        ),
        grid_spec=pltpu.PrefetchScalarGridSpec(
            num_scalar_prefetch=0,
            grid=(1,),
            in_specs=[
                pl.BlockSpec(state.shape, lambda i: (0, 0, 0)),
                pl.BlockSpec(a_tm.shape, lambda i: (0, 0, 0)),
                pl.BlockSpec(la_tm.shape, lambda i: (0, 0, 0)),
                pl.BlockSpec(memory_space=pl.ANY),
                pl.BlockSpec(memory_space=pl.ANY),
                pl.BlockSpec(memory_space=pl.ANY),
                pl.BlockSpec(memory_space=pl.ANY),
            ],
            out_specs=[
                pl.BlockSpec((1, T * B), lambda i: (0, 0)),
                pl.BlockSpec((1, B, H), lambda i: (0, 0, 0)),
                pl.BlockSpec((1, B, H), lambda i: (0, 0, 0)),
            ],
            scratch_shapes=[
                pltpu.VMEM((T * B, d_in), jnp.float32),
                pltpu.VMEM((T * B, G), jnp.float32),
                pltpu.VMEM((L["o_wih"], H), jnp.float32),
                pltpu.VMEM((L["o_whh"] - L["o_wih"], G), jnp.float32),
                pltpu.VMEM((L["o_bl"] - L["o_whh"], G), jnp.float32),
                pltpu.VMEM((B, H), jnp.float32),
                pltpu.VMEM((B, H), jnp.float32),
                pltpu.VMEM((L["epi_rows"], H), jnp.float32),
                pltpu.VMEM((L["main_rows"] - L["o_bl"], G), jnp.float32),
                pltpu.SemaphoreType.DMA,
                pltpu.SemaphoreType.DMA,
                pltpu.SemaphoreType.DMA,
                pltpu.SemaphoreType.DMA,
                pltpu.SemaphoreType.DMA,
                pltpu.SemaphoreType.DMA,
                pltpu.SemaphoreType.DMA,
                pltpu.SemaphoreType.DMA,
                pltpu.SemaphoreType.DMA,
            ],
        ),
        compiler_params=pltpu.CompilerParams(
            dimension_semantics=("arbitrary",),
        ),
        cost_estimate=cost,
    )(state, a_tm, la_tm, h0, c0, epi, main)

    q = q_row.reshape(T, B).T[..., None]
    return q, (h_n, c_n)


# speedup vs baseline: 1.0626x; 1.0371x over previous
import jax
import jax.numpy as jnp
from jax.experimental import pallas as pl
from jax.experimental.pallas import tpu as pltpu


def _rup8(n):
    return (n + 7) & ~7


def _slab_offsets(d_in, h):
    o_w1 = 0
    o_b1 = _rup8(o_w1 + d_in)
    o_wih = _rup8(o_b1 + 1)
    o_whh = o_wih + _rup8(h)
    o_bl = o_whh + _rup8(h)
    main_rows = _rup8(o_bl + 1)
    e_w2 = 0
    e_b2 = _rup8(e_w2 + h)
    e_w3 = _rup8(e_b2 + 1)
    e_b3 = _rup8(e_w3 + 1)
    epi_rows = _rup8(e_b3 + 1)
    return dict(o_w1=o_w1, o_b1=o_b1, o_wih=o_wih, o_whh=o_whh, o_bl=o_bl,
                main_rows=main_rows, e_w2=e_w2, e_b2=e_b2, e_w3=e_w3,
                e_b3=e_b3, epi_rows=epi_rows)


def _fused_kernel(s_ref, a_ref, la_ref, h0_hbm, c0_hbm, epi_hbm, main_hbm,
                  q_ref, hn_ref, cn_ref,
                  xtm_buf, pre_buf, w1_buf, wih_buf, whh_buf,
                  h_buf, c_buf, epi_buf, bl_buf,
                  sem1, sem2, sem3, sem4, sem5, sem6, sem0):
    B, T, sd = s_ref.shape
    ad = a_ref.shape[1]
    lad = la_ref.shape[1]
    d_in = sd + ad + lad
    H = h_buf.shape[1]
    L = _slab_offsets(d_in, H)
    main_rows_pad = L["main_rows"] - L["o_bl"]

    cp1 = pltpu.make_async_copy(
        main_hbm.at[pl.ds(0, L["o_wih"]), pl.ds(0, H)], w1_buf, sem1)
    cp2 = pltpu.make_async_copy(
        main_hbm.at[pl.ds(L["o_wih"], L["o_whh"] - L["o_wih"])], wih_buf, sem2)
    cp3 = pltpu.make_async_copy(
        main_hbm.at[pl.ds(L["o_whh"], L["o_bl"] - L["o_whh"])], whh_buf, sem3)
    cp0 = pltpu.make_async_copy(
        main_hbm.at[pl.ds(L["o_bl"], main_rows_pad)], bl_buf, sem0)
    cp4 = pltpu.make_async_copy(h0_hbm.at[0], h_buf, sem4)
    cp5 = pltpu.make_async_copy(c0_hbm.at[0], c_buf, sem5)
    cp6 = pltpu.make_async_copy(
        epi_hbm.at[pl.ds(0, L["epi_rows"]), pl.ds(0, H)], epi_buf, sem6)
    cp0.start()
    cp1.start()
    cp2.start()
    cp3.start()
    cp4.start()
    cp5.start()
    cp6.start()

    for t in range(T):
        xtm_buf[t * B:(t + 1) * B, 0:sd] = s_ref[:, t, :]
        xtm_buf[t * B:(t + 1) * B, sd:sd + ad] = a_ref[t].T
        xtm_buf[t * B:(t + 1) * B, sd + ad:d_in] = la_ref[t].T

    cp1.wait()
    w1 = w1_buf[0:d_in, :]
    b1 = w1_buf[L["o_b1"]:L["o_b1"] + 1, :]
    a1 = jnp.maximum(
        jnp.dot(xtm_buf[...], w1, preferred_element_type=jnp.float32) + b1,
        0.0)

    cp0.wait()
    cp2.wait()
    pre_buf[...] = jnp.dot(a1, wih_buf[0:H, :],
                           preferred_element_type=jnp.float32) + bl_buf[0:1, :]

    cp3.wait()
    w_hh = whh_buf[0:H, :]

    cp4.wait()
    cp5.wait()
    h = h_buf[...]
    c = c_buf[...]
    hs_steps = []
    for t in range(T):
        gt = pre_buf[t * B:(t + 1) * B, :] + jnp.dot(
            h, w_hh, preferred_element_type=jnp.float32)
        s_if = jax.nn.sigmoid(gt[:, 0:2 * H])
        o_g = jax.nn.sigmoid(gt[:, 3 * H:4 * H])
        g_g = jnp.tanh(gt[:, 2 * H:3 * H])
        c = s_if[:, H:2 * H] * c + s_if[:, 0:H] * g_g
        h = o_g * jnp.tanh(c)
        hs_steps.append(h)

    hn_ref[0] = h
    cn_ref[0] = c

    cp6.wait()
    w2 = epi_buf[0:H, :]
    b2 = epi_buf[L["e_b2"]:L["e_b2"] + 1, :]
    w3r = epi_buf[L["e_w3"]:L["e_w3"] + 1, :]
    b3 = epi_buf[L["e_b3"]:L["e_b3"] + 1, 0:1]
    hs = jnp.concatenate(hs_steps, axis=0)
    a2 = jnp.maximum(
        jnp.dot(hs, w2, preferred_element_type=jnp.float32) + b2, 0.0)
    q_ref[...] = jnp.dot(w3r, a2.T, preferred_element_type=jnp.float32) + b3


def kernel(main, epi, state, action, last_action, h0, c0):
    B, T, _ = state.shape
    H = h0.shape[-1]
    d_in = state.shape[-1] + action.shape[-1] + last_action.shape[-1]
    L = _slab_offsets(d_in, H)
    G = 4 * H

    a_tm = jnp.transpose(action, (1, 2, 0))
    la_tm = jnp.transpose(last_action, (1, 2, 0))

    flops = 2 * T * B * (d_in * H + 2 * H * 4 * H + H * H + H)
    nbytes = 4 * (T * B * d_in + 2 * B * H + L["o_wih"] * H
                  + (L["main_rows"] - L["o_wih"]) * G
                  + L["epi_rows"] * H + T * B + 2 * B * H)
    cost = pl.CostEstimate(flops=flops,
                           transcendentals=3 * T * B * H,
                           bytes_accessed=nbytes)

    q_row, h_n, c_n = pl.pallas_call(
        _fused_kernel,
        out_shape=(
            jax.ShapeDtypeStruct((1, T * B), jnp.float32),
            jax.ShapeDtypeStruct((1, B, H), jnp.float32),
            jax.ShapeDtypeStruct((1, B, H), jnp.float32),
        ),
        grid_spec=pltpu.PrefetchScalarGridSpec(
            num_scalar_prefetch=0,
            grid=(1,),
            in_specs=[
                pl.BlockSpec(state.shape, lambda i: (0, 0, 0)),
                pl.BlockSpec(a_tm.shape, lambda i: (0, 0, 0)),
                pl.BlockSpec(la_tm.shape, lambda i: (0, 0, 0)),
                pl.BlockSpec(memory_space=pl.ANY),
                pl.BlockSpec(memory_space=pl.ANY),
                pl.BlockSpec(memory_space=pl.ANY),
                pl.BlockSpec(memory_space=pl.ANY),
            ],
            out_specs=[
                pl.BlockSpec((1, T * B), lambda i: (0, 0)),
                pl.BlockSpec((1, B, H), lambda i: (0, 0, 0)),
                pl.BlockSpec((1, B, H), lambda i: (0, 0, 0)),
            ],
            scratch_shapes=[
                pltpu.VMEM((T * B, d_in), jnp.float32),
                pltpu.VMEM((T * B, G), jnp.float32),
                pltpu.VMEM((L["o_wih"], H), jnp.float32),
                pltpu.VMEM((L["o_whh"] - L["o_wih"], G), jnp.float32),
                pltpu.VMEM((L["o_bl"] - L["o_whh"], G), jnp.float32),
                pltpu.VMEM((B, H), jnp.float32),
                pltpu.VMEM((B, H), jnp.float32),
                pltpu.VMEM((L["epi_rows"], H), jnp.float32),
                pltpu.VMEM((L["main_rows"] - L["o_bl"], G), jnp.float32),
                pltpu.SemaphoreType.DMA,
                pltpu.SemaphoreType.DMA,
                pltpu.SemaphoreType.DMA,
                pltpu.SemaphoreType.DMA,
                pltpu.SemaphoreType.DMA,
                pltpu.SemaphoreType.DMA,
                pltpu.SemaphoreType.DMA,
            ],
        ),
        compiler_params=pltpu.CompilerParams(
            dimension_semantics=("arbitrary",),
        ),
        cost_estimate=cost,
    )(state, a_tm, la_tm, h0, c0, epi, main)

    q = q_row.reshape(T, B).T[..., None]
    return q, (h_n, c_n)
